# baseline (speedup 1.0000x reference)
import jax
import jax.numpy as jnp
from jax import lax
from jax.experimental import pallas as pl
from jax.experimental.pallas import tpu as pltpu
from jax.experimental.pallas import tpu_sc as plsc

_NUM_SAMPLES = 32
_C = 3
_T = 128
_ROW = 256 * 256
_NW = 32
_ROWS_PER_W = (_C * _NUM_SAMPLES) // _NW
_CH = 32768
_NCH = _ROW // _CH


def _body(x_ref, o_ref, buf, sem):
    w = lax.axis_index("s") * 2 + lax.axis_index("c")
    for k in range(_ROWS_PER_W):
        r = w * _ROWS_PER_W + k
        c = r // _NUM_SAMPLES
        j = r % _NUM_SAMPLES
        t = (j * (_T - 1)) // (_NUM_SAMPLES - 1)
        src = c * _T + t
        for h in range(_NCH):
            pltpu.async_copy(
                x_ref.at[src, pl.ds(h * _CH, _CH)], buf, sem
            ).wait()
            pltpu.async_copy(
                buf, o_ref.at[r, pl.ds(h * _CH, _CH)], sem
            ).wait()


def kernel(x):
    xf = x.reshape(_C * _T, _ROW)
    out = pl.kernel(
        _body,
        out_type=jax.ShapeDtypeStruct((_C * _NUM_SAMPLES, _ROW), jnp.float32),
        mesh=plsc.VectorSubcoreMesh(
            core_axis_name="c", subcore_axis_name="s",
            num_cores=2, num_subcores=16,
        ),
        scratch_types=[
            pltpu.VMEM((_CH,), jnp.float32),
            pltpu.SemaphoreType.DMA,
        ],
    )(xf)
    return out.reshape(_C, _NUM_SAMPLES, 256, 256)


# speedup vs baseline: 1.0335x; 1.0335x over previous
import jax
import jax.numpy as jnp
from jax import lax
from jax.experimental import pallas as pl
from jax.experimental.pallas import tpu as pltpu
from jax.experimental.pallas import tpu_sc as plsc

_NUM_SAMPLES = 32
_C = 3
_T = 128
_ROW = 256 * 256
_NW = 32
_ROWS_PER_W = (_C * _NUM_SAMPLES) // _NW
_CH = 16384
_NCH = _ROW // _CH
_NBUF = 7
_NCHUNKS = _ROWS_PER_W * _NCH


def _body(x_ref, o_ref, buf, rsem, wsem):
    w = lax.axis_index("s") * 2 + lax.axis_index("c")
    chunks = []
    for k in range(_ROWS_PER_W):
        r = w * _ROWS_PER_W + k
        c = r // _NUM_SAMPLES
        j = r % _NUM_SAMPLES
        t = (j * (_T - 1)) // (_NUM_SAMPLES - 1)
        src = c * _T + t
        for h in range(_NCH):
            chunks.append((src, r, h * _CH))

    def read(i):
        src, _, off = chunks[i]
        b = i % _NBUF
        return pltpu.async_copy(
            x_ref.at[src, pl.ds(off, _CH)],
            buf.at[pl.ds(b * _CH, _CH)], rsem.at[b])

    reads = [None] * _NCHUNKS
    writes = [None] * _NCHUNKS
    for i in range(min(_NBUF, _NCHUNKS)):
        reads[i] = read(i)
    for i in range(_NCHUNKS):
        _, r, off = chunks[i]
        b = i % _NBUF
        reads[i].wait()
        writes[i] = pltpu.async_copy(
            buf.at[pl.ds(b * _CH, _CH)],
            o_ref.at[r, pl.ds(off, _CH)], wsem.at[b])
        ni = i + _NBUF
        if ni < _NCHUNKS:
            writes[i].wait()
            writes[i] = None
            reads[ni] = read(ni)
    for wr in writes:
        if wr is not None:
            wr.wait()


def kernel(x):
    xf = x.reshape(_C * _T, _ROW)
    out = pl.kernel(
        _body,
        out_type=jax.ShapeDtypeStruct((_C * _NUM_SAMPLES, _ROW), jnp.float32),
        mesh=plsc.VectorSubcoreMesh(
            core_axis_name="c", subcore_axis_name="s",
            num_cores=2, num_subcores=16,
        ),
        scratch_types=[
            pltpu.VMEM((_NBUF * _CH,), jnp.float32),
            pltpu.SemaphoreType.DMA((_NBUF,)),
            pltpu.SemaphoreType.DMA((_NBUF,)),
        ],
    )(xf)
    return out.reshape(_C, _NUM_SAMPLES, 256, 256)
